# baseline (speedup 1.0000x reference)
import functools

import jax
import jax.numpy as jnp
from jax import lax
from jax.experimental import pallas as pl
from jax.experimental.pallas import tpu as pltpu
from jax.experimental.pallas import tpu_sc as plsc

D = 128
N_AGT = 2048
N_CTX = 2048
K = 64
BA = 64
EPS = 1e-5
TH2 = 9.0
PADV = 1.0e9
NCORE = 2
NSUB = 16
NW = NCORE * NSUB
APW = N_AGT // NW
LANES = 16


def _gn(x, g, b):
    m = jnp.mean(x, axis=-1, keepdims=True)
    v = jnp.mean((x - m) * (x - m), axis=-1, keepdims=True)
    return (x - m) * jax.lax.rsqrt(v + EPS) * g + b


def _ctxc_body(ctx_ref, ct_ref, out_ref):
    out_ref[...] = jnp.dot(ctx_ref[...], ct_ref[...],
                           preferred_element_type=jnp.float32)


def _run_ctxc(ctx, ct):
    return pl.pallas_call(
        _ctxc_body,
        grid=(8,),
        in_specs=[
            pl.BlockSpec((N_CTX // 8, D), lambda i: (i, 0)),
            pl.BlockSpec((D, D), lambda i: (0, 0)),
        ],
        out_specs=pl.BlockSpec((N_CTX // 8, D), lambda i: (i, 0)),
        out_shape=jax.ShapeDtypeStruct((N_CTX, D), jnp.float32),
    )(ctx, ct)


GB = 128
NQ = 4


PB = 1536
BPW = APW // BA
PPW = BPW * PB


def _sc_body(cxy, axy, ctxc, dx_out, dy_out, sid_out, gath_out,
             cx_v, cy_v, ax_v, ay_v, ldx, ldy, lidx, lsid, rows_v, spc,
             sg, ss):
    c = lax.axis_index("c")
    s = lax.axis_index("s")
    wid = s * NCORE + c
    base_a = wid * APW
    base_p = wid * PPW

    @pl.when(s == 0)
    def _stage():
        pltpu.sync_copy(ctxc, spc)

    pltpu.sync_copy(cxy.at[0], cx_v)
    pltpu.sync_copy(cxy.at[1], cy_v)
    pltpu.sync_copy(axy.at[0, pl.ds(base_a, APW)], ax_v.at[pl.ds(0, APW)])
    pltpu.sync_copy(axy.at[1, pl.ds(base_a, APW)], ay_v.at[pl.ds(0, APW)])

    for blk in range(BPW):
        pbase = blk * PB

        def initslots(t, carry):
            off = pl.multiple_of(pbase + t * LANES, LANES)
            ldx[pl.ds(off, LANES)] = jnp.full((LANES,), PADV, jnp.float32)
            ldy[pl.ds(off, LANES)] = jnp.full((LANES,), PADV, jnp.float32)
            lidx[pl.ds(off, LANES)] = jnp.zeros((LANES,), jnp.int32)
            lsid[pl.ds(off, LANES)] = jnp.full((LANES,), -1.0, jnp.float32)
            return carry

        lax.fori_loop(0, PB // LANES, initslots, 0, unroll=4)

        def agent_body(a_loc, cnt_blk):
            ax = ax_v[pl.ds(blk * BA + a_loc, LANES)][0]
            ay = ay_v[pl.ds(blk * BA + a_loc, LANES)][0]
            sidv = jnp.full((LANES,), a_loc, jnp.int32).astype(jnp.float32)

            def chunk(j, cnt):
                off = pl.multiple_of(j * LANES, LANES)
                cxv = cx_v[pl.ds(off, LANES)]
                cyv = cy_v[pl.ds(off, LANES)]
                dxv = ax - cxv
                dyv = ay - cyv
                d2 = dxv * dxv + dyv * dyv
                m = d2 <= TH2
                npop = plsc.all_reduce_population_count(m)[0]

                @pl.when(npop > 0)
                def _emit():
                    mi = m.astype(jnp.int32)
                    incl = plsc.cumsum(mi)
                    slot = cnt + incl - 1
                    ok = jnp.logical_and(m, slot < PB)
                    slotc = jnp.minimum(jnp.maximum(slot, 0),
                                        PB - 1) + pbase
                    iv = off + lax.iota(jnp.int32, LANES)
                    plsc.store_scatter(ldx, [slotc], dxv, mask=ok)
                    plsc.store_scatter(ldy, [slotc], dyv, mask=ok)
                    plsc.store_scatter(lidx, [slotc], iv, mask=ok)
                    plsc.store_scatter(lsid, [slotc], sidv, mask=ok)

                return cnt + npop

            return lax.fori_loop(0, N_CTX // LANES, chunk, cnt_blk,
                                 unroll=4)

        lax.fori_loop(0, BA, agent_body, 0)

    pltpu.sync_copy(ldx, dx_out.at[pl.ds(base_p, PPW)])
    pltpu.sync_copy(ldy, dy_out.at[pl.ds(base_p, PPW)])
    pltpu.sync_copy(lsid, sid_out.at[pl.ds(base_p, PPW)])

    plsc.subcore_barrier()
    nsteps = PPW // (GB * NQ)

    def gstep(hh, carry):
        b0 = hh * NQ
        for q in range(NQ):
            pltpu.async_copy(
                spc.at[lidx.at[pl.ds((b0 + q) * GB, GB)]],
                rows_v.at[pl.ds(q * GB, GB)], sg)
        for q in range(NQ):
            pltpu.make_async_copy(
                spc.at[lidx.at[pl.ds((b0 + q) * GB, GB)]],
                rows_v.at[pl.ds(q * GB, GB)], sg).wait()
        for q in range(NQ):
            pltpu.async_copy(
                rows_v.at[pl.ds(q * GB, GB)],
                gath_out.at[pl.ds(base_p + (b0 + q) * GB, GB)], ss)
        for q in range(NQ):
            pltpu.make_async_copy(
                rows_v.at[pl.ds(q * GB, GB)],
                gath_out.at[pl.ds(base_p + (b0 + q) * GB, GB)], ss).wait()
        return carry

    lax.fori_loop(0, nsteps, gstep, 0)


NBLK = N_AGT // BA
NPAIR = NBLK * PB


def _run_sc(cxy, axy, ctxc):
    mesh = plsc.VectorSubcoreMesh(
        core_axis_name="c", subcore_axis_name="s",
        num_cores=NCORE, num_subcores=NSUB)
    fn = pl.kernel(
        _sc_body,
        out_type=[
            jax.ShapeDtypeStruct((NPAIR,), jnp.float32),
            jax.ShapeDtypeStruct((NPAIR,), jnp.float32),
            jax.ShapeDtypeStruct((NPAIR,), jnp.float32),
            jax.ShapeDtypeStruct((NPAIR, D), jnp.float32),
        ],
        mesh=mesh,
        compiler_params=pltpu.CompilerParams(needs_layout_passes=False),
        scratch_types=[
            pltpu.VMEM((N_CTX,), jnp.float32),
            pltpu.VMEM((N_CTX,), jnp.float32),
            pltpu.VMEM((APW + LANES,), jnp.float32),
            pltpu.VMEM((APW + LANES,), jnp.float32),
            pltpu.VMEM((PPW,), jnp.float32),
            pltpu.VMEM((PPW,), jnp.float32),
            pltpu.VMEM((PPW,), jnp.int32),
            pltpu.VMEM((PPW,), jnp.float32),
            pltpu.VMEM((GB * NQ, D), jnp.float32),
            pltpu.VMEM_SHARED((N_CTX, D), jnp.float32),
            pltpu.SemaphoreType.DMA,
            pltpu.SemaphoreType.DMA,
        ],
    )
    return fn(cxy, axy, ctxc)


def _main_body(ag_ref, dx_ref, dy_ref, sid_ref, gath_ref,
               wqt, gq, beq, w1x, w1y, b1, wd2t, gd2, bed2,
               wc1at, gc1, bec1, wc2t, wbt, wat,
               gn_, ben, wlt, gl, bel, out_ref):
    ag = ag_ref[...]
    q = jax.nn.relu(_gn(jnp.dot(ag, wqt[...],
                                preferred_element_type=jnp.float32),
                        gq[...], beq[...]))
    qb = jnp.dot(q, wbt[...], preferred_element_type=jnp.float32)
    a0 = jnp.dot(ag, wat[...], preferred_element_type=jnp.float32)

    dxc = dx_ref[...].reshape(PB)[:, None]
    dyc = dy_ref[...].reshape(PB)[:, None]
    sid = sid_ref[...].reshape(PB)[:, None]
    aidx = lax.broadcasted_iota(jnp.int32, (1, BA), 1)
    oh = jnp.where(sid.astype(jnp.int32) == aidx, 1.0, 0.0)

    d1f = jax.nn.relu(dxc * w1x[...] + dyc * w1y[...] + b1[...])
    d2f = jax.nn.relu(_gn(jnp.dot(d1f, wd2t[...],
                                  preferred_element_type=jnp.float32),
                          gd2[...], bed2[...]))
    xf = jnp.dot(d2f, wc1at[...], preferred_element_type=jnp.float32)
    xf = xf + jnp.dot(oh, qb, preferred_element_type=jnp.float32) \
        + gath_ref[...]
    hf = jax.nn.relu(_gn(xf, gc1[...], bec1[...]))
    of = jnp.dot(hf, wc2t[...], preferred_element_type=jnp.float32)
    msg = lax.dot_general(oh, of, (((0,), (0,)), ((), ())),
                          preferred_element_type=jnp.float32)

    a = a0 + msg
    a = jax.nn.relu(_gn(a, gn_[...], ben[...]))
    a = _gn(jnp.dot(a, wlt[...], preferred_element_type=jnp.float32),
            gl[...], bel[...])
    out_ref[...] = jax.nn.relu(a + ag)


def _run_main(ag, dx, dy, sid, gath, *ws):
    wspecs = []
    for w in ws:
        shp = w.shape
        wspecs.append(pl.BlockSpec(shp, lambda i, _s=len(shp): (0,) * _s))
    return pl.pallas_call(
        _main_body,
        grid=(NBLK,),
        in_specs=[
            pl.BlockSpec((BA, D), lambda i: (i, 0)),
            pl.BlockSpec((1, 1, PB), lambda i: (i, 0, 0)),
            pl.BlockSpec((1, 1, PB), lambda i: (i, 0, 0)),
            pl.BlockSpec((1, 1, PB), lambda i: (i, 0, 0)),
            pl.BlockSpec((PB, D), lambda i: (i, 0)),
        ] + wspecs,
        out_specs=pl.BlockSpec((BA, D), lambda i: (i, 0)),
        out_shape=jax.ShapeDtypeStruct((N_AGT, D), jnp.float32),
    )(ag, dx, dy, sid, gath, *ws)


def kernel(agts, ctx, agt_ctrs, ctx_ctrs, W_d1, b_d1, W_d2, g_d2, be_d2,
           W_q, g_q, be_q, W_c1, g_c1, be_c1, W_c2, W_a, g_n, be_n,
           W_l, g_l, be_l):
    r1 = lambda v: v.reshape(1, D)
    A = W_c1[:, :D]
    B = W_c1[:, D:2 * D]
    C = W_c1[:, 2 * D:]

    ctxc = _run_ctxc(ctx, C.T)
    cxy = jnp.transpose(ctx_ctrs)
    axy = jnp.transpose(agt_ctrs)
    dxf, dyf, sidf, gath = _run_sc(cxy, axy, ctxc)
    dx = dxf.reshape(NBLK, 1, PB)
    dy = dyf.reshape(NBLK, 1, PB)
    sid = sidf.reshape(NBLK, 1, PB)

    ws = (W_q.T, r1(g_q), r1(be_q),
          r1(W_d1[:, 0]), r1(W_d1[:, 1]), r1(b_d1),
          W_d2.T, r1(g_d2), r1(be_d2),
          A.T, r1(g_c1), r1(be_c1),
          W_c2.T, B.T, W_a.T,
          r1(g_n), r1(be_n),
          W_l.T, r1(g_l), r1(be_l))
    return _run_main(agts, dx, dy, sid, gath, *ws)


# speedup vs baseline: 1.6295x; 1.6295x over previous
import functools

import jax
import jax.numpy as jnp
from jax import lax
from jax.experimental import pallas as pl
from jax.experimental.pallas import tpu as pltpu
from jax.experimental.pallas import tpu_sc as plsc

D = 128
N_AGT = 2048
N_CTX = 2048
K = 64
BA = 64
EPS = 1e-5
TH2 = 9.0
PADV = 1.0e9
NCORE = 2
NSUB = 16
NW = NCORE * NSUB
APW = N_AGT // NW
LANES = 16


def _gn(x, g, b):
    m = jnp.mean(x, axis=-1, keepdims=True)
    v = jnp.mean((x - m) * (x - m), axis=-1, keepdims=True)
    return (x - m) * jax.lax.rsqrt(v + EPS) * g + b


def _ctxc_body(ctx_ref, ct_ref, out_ref):
    out_ref[...] = jnp.dot(ctx_ref[...], ct_ref[...],
                           preferred_element_type=jnp.float32)


def _run_ctxc(ctx, ct):
    return pl.pallas_call(
        _ctxc_body,
        grid=(8,),
        in_specs=[
            pl.BlockSpec((N_CTX // 8, D), lambda i: (i, 0)),
            pl.BlockSpec((D, D), lambda i: (0, 0)),
        ],
        out_specs=pl.BlockSpec((N_CTX // 8, D), lambda i: (i, 0)),
        out_shape=jax.ShapeDtypeStruct((N_CTX, D), jnp.float32),
    )(ctx, ct)


GB = 128
NQ = 4


PB = 1536
BPW = APW // BA
PPW = BPW * PB


def _sc_body(cxy, axy, ctxc, dx_out, dy_out, sid_out, gath_out,
             cx_v, cy_v, ax_v, ay_v, ldx, ldy, lidx, lsid,
             ldxK, ldyK, lidxK, cntb, rows_v, spc, sg, ss):
    c = lax.axis_index("c")
    s = lax.axis_index("s")
    wid = s * NCORE + c
    base_a = wid * APW
    base_p = wid * PPW

    @pl.when(s == 0)
    def _stage():
        pltpu.sync_copy(ctxc, spc)

    pltpu.sync_copy(cxy.at[0], cx_v)
    pltpu.sync_copy(cxy.at[1], cy_v)
    pltpu.sync_copy(axy.at[0, pl.ds(base_a, APW)], ax_v.at[pl.ds(0, APW)])
    pltpu.sync_copy(axy.at[1, pl.ds(base_a, APW)], ay_v.at[pl.ds(0, APW)])

    iota16 = lax.iota(jnp.int32, LANES)
    for g in range(APW // LANES):
        base16 = (g * LANES + iota16) * K
        ax16 = ax_v[pl.ds(g * LANES, LANES)]
        ay16 = ay_v[pl.ds(g * LANES, LANES)]

        def cbody(j, cnt16):
            for kk in range(LANES):
                cid = j * LANES + kk
                sel = jnp.full((LANES,), cid, jnp.int32)
                bcx = plsc.load_gather(cx_v, [sel])
                bcy = plsc.load_gather(cy_v, [sel])
                dxv = ax16 - bcx
                dyv = ay16 - bcy
                d2 = dxv * dxv + dyv * dyv
                m = d2 <= TH2
                slotv = base16 + jnp.minimum(cnt16, K - 1)
                iv = jnp.full((LANES,), cid, jnp.int32)
                plsc.store_scatter(ldxK, [slotv], dxv, mask=m)
                plsc.store_scatter(ldyK, [slotv], dyv, mask=m)
                plsc.store_scatter(lidxK, [slotv], iv, mask=m)
                cnt16 = cnt16 + m.astype(jnp.int32)
            return cnt16

        cnt16 = lax.fori_loop(0, N_CTX // LANES, cbody,
                              jnp.zeros((LANES,), jnp.int32))
        cntb[pl.ds(g * LANES, LANES)] = jnp.minimum(cnt16, K)

    def initslots(t, carry):
        off = pl.multiple_of(t * LANES, LANES)
        ldx[pl.ds(off, LANES)] = jnp.full((LANES,), PADV, jnp.float32)
        ldy[pl.ds(off, LANES)] = jnp.full((LANES,), PADV, jnp.float32)
        lidx[pl.ds(off, LANES)] = jnp.zeros((LANES,), jnp.int32)
        lsid[pl.ds(off, LANES)] = jnp.full((LANES,), -1.0, jnp.float32)
        return carry

    lax.fori_loop(0, PB // LANES, initslots, 0, unroll=4)

    def compact(a, off):
        cnt_a = cntb[pl.ds(a, LANES)][0]
        sidv = jnp.full((LANES,), a, jnp.int32).astype(jnp.float32)
        for t in range(K // LANES):
            src_dx = ldxK[pl.ds(a * K + t * LANES, LANES)]
            src_dy = ldyK[pl.ds(a * K + t * LANES, LANES)]
            src_ix = lidxK[pl.ds(a * K + t * LANES, LANES)]
            pos = t * LANES + iota16
            okv = jnp.logical_and(pos < cnt_a, (off + pos) < PB)
            dstv = jnp.minimum(off + pos, PB - 1)
            plsc.store_scatter(ldx, [dstv], src_dx, mask=okv)
            plsc.store_scatter(ldy, [dstv], src_dy, mask=okv)
            plsc.store_scatter(lidx, [dstv], src_ix, mask=okv)
            plsc.store_scatter(lsid, [dstv], sidv, mask=okv)
        return off + cnt_a

    lax.fori_loop(0, APW, compact, 0)

    pltpu.sync_copy(ldx, dx_out.at[pl.ds(base_p, PPW)])
    pltpu.sync_copy(ldy, dy_out.at[pl.ds(base_p, PPW)])
    pltpu.sync_copy(lsid, sid_out.at[pl.ds(base_p, PPW)])

    plsc.subcore_barrier()
    nsteps = PPW // (GB * NQ)

    def gstep(hh, carry):
        b0 = hh * NQ
        for q in range(NQ):
            pltpu.async_copy(
                spc.at[lidx.at[pl.ds((b0 + q) * GB, GB)]],
                rows_v.at[pl.ds(q * GB, GB)], sg)
        for q in range(NQ):
            pltpu.make_async_copy(
                spc.at[lidx.at[pl.ds((b0 + q) * GB, GB)]],
                rows_v.at[pl.ds(q * GB, GB)], sg).wait()
        for q in range(NQ):
            pltpu.async_copy(
                rows_v.at[pl.ds(q * GB, GB)],
                gath_out.at[pl.ds(base_p + (b0 + q) * GB, GB)], ss)
        for q in range(NQ):
            pltpu.make_async_copy(
                rows_v.at[pl.ds(q * GB, GB)],
                gath_out.at[pl.ds(base_p + (b0 + q) * GB, GB)], ss).wait()
        return carry

    lax.fori_loop(0, nsteps, gstep, 0)


NBLK = N_AGT // BA
NPAIR = NBLK * PB


def _run_sc(cxy, axy, ctxc):
    mesh = plsc.VectorSubcoreMesh(
        core_axis_name="c", subcore_axis_name="s",
        num_cores=NCORE, num_subcores=NSUB)
    fn = pl.kernel(
        _sc_body,
        out_type=[
            jax.ShapeDtypeStruct((NPAIR,), jnp.float32),
            jax.ShapeDtypeStruct((NPAIR,), jnp.float32),
            jax.ShapeDtypeStruct((NPAIR,), jnp.float32),
            jax.ShapeDtypeStruct((NPAIR, D), jnp.float32),
        ],
        mesh=mesh,
        compiler_params=pltpu.CompilerParams(needs_layout_passes=False),
        scratch_types=[
            pltpu.VMEM((N_CTX,), jnp.float32),
            pltpu.VMEM((N_CTX,), jnp.float32),
            pltpu.VMEM((APW + LANES,), jnp.float32),
            pltpu.VMEM((APW + LANES,), jnp.float32),
            pltpu.VMEM((PPW,), jnp.float32),
            pltpu.VMEM((PPW,), jnp.float32),
            pltpu.VMEM((PPW,), jnp.int32),
            pltpu.VMEM((PPW,), jnp.float32),
            pltpu.VMEM((APW * K,), jnp.float32),
            pltpu.VMEM((APW * K,), jnp.float32),
            pltpu.VMEM((APW * K,), jnp.int32),
            pltpu.VMEM((APW + LANES,), jnp.int32),
            pltpu.VMEM((GB * NQ, D), jnp.float32),
            pltpu.VMEM_SHARED((N_CTX, D), jnp.float32),
            pltpu.SemaphoreType.DMA,
            pltpu.SemaphoreType.DMA,
        ],
    )
    return fn(cxy, axy, ctxc)


def _main_body(ag_ref, dx_ref, dy_ref, sid_ref, gath_ref,
               wqt, gq, beq, w1x, w1y, b1, wd2t, gd2, bed2,
               wc1at, gc1, bec1, wc2t, wbt, wat,
               gn_, ben, wlt, gl, bel, out_ref):
    ag = ag_ref[...]
    q = jax.nn.relu(_gn(jnp.dot(ag, wqt[...],
                                preferred_element_type=jnp.float32),
                        gq[...], beq[...]))
    qb = jnp.dot(q, wbt[...], preferred_element_type=jnp.float32)
    a0 = jnp.dot(ag, wat[...], preferred_element_type=jnp.float32)

    dxc = dx_ref[...].reshape(PB)[:, None]
    dyc = dy_ref[...].reshape(PB)[:, None]
    sid = sid_ref[...].reshape(PB)[:, None]
    aidx = lax.broadcasted_iota(jnp.int32, (1, BA), 1)
    oh = jnp.where(sid.astype(jnp.int32) == aidx, 1.0, 0.0)

    d1f = jax.nn.relu(dxc * w1x[...] + dyc * w1y[...] + b1[...])
    d2f = jax.nn.relu(_gn(jnp.dot(d1f, wd2t[...],
                                  preferred_element_type=jnp.float32),
                          gd2[...], bed2[...]))
    xf = jnp.dot(d2f, wc1at[...], preferred_element_type=jnp.float32)
    xf = xf + jnp.dot(oh, qb, preferred_element_type=jnp.float32) \
        + gath_ref[...]
    hf = jax.nn.relu(_gn(xf, gc1[...], bec1[...]))
    of = jnp.dot(hf, wc2t[...], preferred_element_type=jnp.float32)
    msg = lax.dot_general(oh, of, (((0,), (0,)), ((), ())),
                          preferred_element_type=jnp.float32)

    a = a0 + msg
    a = jax.nn.relu(_gn(a, gn_[...], ben[...]))
    a = _gn(jnp.dot(a, wlt[...], preferred_element_type=jnp.float32),
            gl[...], bel[...])
    out_ref[...] = jax.nn.relu(a + ag)


def _run_main(ag, dx, dy, sid, gath, *ws):
    wspecs = []
    for w in ws:
        shp = w.shape
        wspecs.append(pl.BlockSpec(shp, lambda i, _s=len(shp): (0,) * _s))
    return pl.pallas_call(
        _main_body,
        grid=(NBLK,),
        in_specs=[
            pl.BlockSpec((BA, D), lambda i: (i, 0)),
            pl.BlockSpec((1, 1, PB), lambda i: (i, 0, 0)),
            pl.BlockSpec((1, 1, PB), lambda i: (i, 0, 0)),
            pl.BlockSpec((1, 1, PB), lambda i: (i, 0, 0)),
            pl.BlockSpec((PB, D), lambda i: (i, 0)),
        ] + wspecs,
        out_specs=pl.BlockSpec((BA, D), lambda i: (i, 0)),
        out_shape=jax.ShapeDtypeStruct((N_AGT, D), jnp.float32),
    )(ag, dx, dy, sid, gath, *ws)


def kernel(agts, ctx, agt_ctrs, ctx_ctrs, W_d1, b_d1, W_d2, g_d2, be_d2,
           W_q, g_q, be_q, W_c1, g_c1, be_c1, W_c2, W_a, g_n, be_n,
           W_l, g_l, be_l):
    r1 = lambda v: v.reshape(1, D)
    A = W_c1[:, :D]
    B = W_c1[:, D:2 * D]
    C = W_c1[:, 2 * D:]

    ctxc = _run_ctxc(ctx, C.T)
    cxy = jnp.transpose(ctx_ctrs)
    axy = jnp.transpose(agt_ctrs)
    dxf, dyf, sidf, gath = _run_sc(cxy, axy, ctxc)
    dx = dxf.reshape(NBLK, 1, PB)
    dy = dyf.reshape(NBLK, 1, PB)
    sid = sidf.reshape(NBLK, 1, PB)

    ws = (W_q.T, r1(g_q), r1(be_q),
          r1(W_d1[:, 0]), r1(W_d1[:, 1]), r1(b_d1),
          W_d2.T, r1(g_d2), r1(be_d2),
          A.T, r1(g_c1), r1(be_c1),
          W_c2.T, B.T, W_a.T,
          r1(g_n), r1(be_n),
          W_l.T, r1(g_l), r1(be_l))
    return _run_main(agts, dx, dy, sid, gath, *ws)
